# baseline (speedup 1.0000x reference)
import functools

import jax
import jax.numpy as jnp
from jax import lax
from jax.experimental import pallas as pl
from jax.experimental.pallas import tpu as pltpu
from jax.experimental.pallas import tpu_sc as plsc


_NC = 2
_NS = 16
_ZR = 512
_BLK = 16384


def _copy_body(x_ref, o_ref):
    o_ref[...] = x_ref[...]


@functools.lru_cache(maxsize=None)
def _make_sc_zero(m, dtype_name):
    dtype = jnp.dtype(dtype_name)
    nw = _NC * _NS
    rpw = m // nw
    nz = rpw // _ZR
    mesh = plsc.VectorSubcoreMesh(
        core_axis_name="c", subcore_axis_name="s",
        num_cores=_NC, num_subcores=_NS,
    )

    @functools.partial(
        pl.kernel,
        out_type=jax.ShapeDtypeStruct((m, 128), dtype),
        mesh=mesh,
        scratch_types=[
            pltpu.VMEM((_ZR, 128), dtype),
            pltpu.SemaphoreType.DMA,
        ],
    )
    def sc_zero(o_hbm, zbuf, zsem):
        wid = lax.axis_index("s") * _NC + lax.axis_index("c")
        base = wid * rpw
        zero16 = jnp.zeros((16,), dtype)

        def zrow(i, carry):
            for j in range(8):
                zbuf[i, pl.ds(16 * j, 16)] = zero16
            return carry

        lax.fori_loop(0, _ZR, zrow, 0)

        zcopies = [
            pltpu.make_async_copy(
                zbuf,
                o_hbm.at[pl.ds(base + k * _ZR, _ZR), :],
                zsem,
            )
            for k in range(nz)
        ]
        for zc in zcopies:
            zc.start()
        for zc in zcopies:
            zc.wait()

    return sc_zero


def kernel(psi):
    rows, cols = psi.shape
    m = (rows * cols) // 128
    flat = psi.reshape(m, 128)
    nb = m // _BLK

    top = pl.pallas_call(
        _copy_body,
        grid=(nb,),
        in_specs=[pl.BlockSpec((_BLK, 128), lambda i: (i, 0))],
        out_specs=pl.BlockSpec((_BLK, 128), lambda i: (i, 0)),
        out_shape=jax.ShapeDtypeStruct((m, 128), psi.dtype),
    )(flat)

    bot = _make_sc_zero(m, psi.dtype.name)()

    out = jnp.concatenate([top, bot], axis=0)
    return out.reshape(2 * rows, cols)


# speedup vs baseline: 1.0944x; 1.0944x over previous
import jax
import jax.numpy as jnp
from jax.experimental import pallas as pl


_BLK = 16384


def kernel(psi):
    rows, cols = psi.shape
    m = (rows * cols) // 128
    flat = psi.reshape(m, 128)
    nb = m // _BLK

    def body(x_ref, o_ref):
        i = pl.program_id(0)

        @pl.when(i < nb)
        def _copy():
            o_ref[...] = x_ref[...]

        @pl.when(i >= nb)
        def _zero():
            o_ref[...] = jnp.zeros_like(o_ref)

    out = pl.pallas_call(
        body,
        grid=(2 * nb,),
        in_specs=[pl.BlockSpec((_BLK, 128), lambda i: (jnp.minimum(i, nb - 1), 0))],
        out_specs=pl.BlockSpec((_BLK, 128), lambda i: (i, 0)),
        out_shape=jax.ShapeDtypeStruct((2 * m, 128), psi.dtype),
    )(flat)

    return out.reshape(2 * rows, cols)


# speedup vs baseline: 1.2933x; 1.1817x over previous
import jax
import jax.numpy as jnp
from jax.experimental import pallas as pl


_BLKN = 16384


def kernel(psi):
    rows, cols = psi.shape
    nb = rows // _BLKN

    def body(x_ref, o_ref):
        i = pl.program_id(0)

        @pl.when(i < nb)
        def _copy():
            o_ref[...] = x_ref[...]

        @pl.when(i >= nb)
        def _zero():
            o_ref[...] = jnp.zeros_like(o_ref)

    return pl.pallas_call(
        body,
        grid=(2 * nb,),
        in_specs=[pl.BlockSpec((_BLKN, cols), lambda i: (jnp.minimum(i, nb - 1), 0))],
        out_specs=pl.BlockSpec((_BLKN, cols), lambda i: (i, 0)),
        out_shape=jax.ShapeDtypeStruct((2 * rows, cols), psi.dtype),
    )(psi)
